# baseline (speedup 1.0000x reference)
import jax
import jax.numpy as jnp
from jax.experimental import pallas as pl


def _double_block(x_ref, o_ref):
    o_ref[...] = x_ref[...] + x_ref[...]


def kernel(x, edge_index):
    n, d = x.shape
    block_rows = 1000
    grid = (n // block_rows,)
    return pl.pallas_call(
        _double_block,
        grid=grid,
        in_specs=[pl.BlockSpec((block_rows, d), lambda i: (i, 0))],
        out_specs=pl.BlockSpec((block_rows, d), lambda i: (i, 0)),
        out_shape=jax.ShapeDtypeStruct((n, d), x.dtype),
    )(x)


# speedup vs baseline: 1.6696x; 1.6696x over previous
import jax
import jax.numpy as jnp
from jax.experimental import pallas as pl


def _double_block(x_ref, o_ref):
    o_ref[...] = x_ref[...] + x_ref[...]


def kernel(x, edge_index):
    n, d = x.shape
    return pl.pallas_call(
        _double_block,
        out_shape=jax.ShapeDtypeStruct((n, d), x.dtype),
    )(x)


# speedup vs baseline: 2.0765x; 1.2437x over previous
import jax
import jax.numpy as jnp
from jax.experimental import pallas as pl


def _double_block(x_ref, o_ref):
    o_ref[...] = x_ref[...] + x_ref[...]


def kernel(x, edge_index):
    n, d = x.shape
    block_rows = 5000
    grid = (n // block_rows,)
    return pl.pallas_call(
        _double_block,
        grid=grid,
        in_specs=[pl.BlockSpec((block_rows, d), lambda i: (i, 0))],
        out_specs=pl.BlockSpec((block_rows, d), lambda i: (i, 0)),
        out_shape=jax.ShapeDtypeStruct((n, d), x.dtype),
    )(x)
